# Initial kernel scaffold; baseline (speedup 1.0000x reference)
import jax
import jax.numpy as jnp
from jax.experimental import pallas as pl


def kernel(x, p):
    raise NotImplementedError("write your pallas kernel here")


# speedup vs baseline: 2.0272x; 2.0272x over previous
import functools

import jax
import jax.numpy as jnp
from jax.experimental import pallas as pl
from jax.experimental.pallas import tpu as pltpu

_EPS = 1e-6


def _sum_pow_kernel(x_ref, p_ref, s_ref):
    xm = jnp.maximum(x_ref[...], _EPS)
    xp = jnp.exp2(jnp.log2(xm) * p_ref[...])
    s_ref[...] = jnp.sum(xp, axis=-1, keepdims=True)


def _finalize_kernel(s_ref, p_ref, o_ref, *, inv_hw):
    m = s_ref[...] * inv_hw
    invp = 1.0 / p_ref[...]
    o_ref[...] = jnp.exp2(jnp.log2(m) * invp)


def kernel(x, p):
    N, C, H, W = x.shape
    HW = H * W
    NC = N * C

    x2 = x.reshape(NC, HW).astype(jnp.float32)
    pf = p.astype(jnp.float32)

    rt = 512
    while rt > 8 and (C % rt or NC % rt):
        rt //= 2
    nrep = C // rt

    p_col = pf.reshape(C, 1)
    grid_a = (NC // rt,)
    s = pl.pallas_call(
        _sum_pow_kernel,
        out_shape=jax.ShapeDtypeStruct((NC, 1), jnp.float32),
        grid=grid_a,
        in_specs=[
            pl.BlockSpec((rt, HW), lambda r: (r, 0)),
            pl.BlockSpec((rt, 1), lambda r: (r % nrep, 0)),
        ],
        out_specs=pl.BlockSpec((rt, 1), lambda r: (r, 0)),
        compiler_params=pltpu.CompilerParams(
            dimension_semantics=("parallel",)),
    )(x2, p_col)

    cb = C // 128
    s3 = s.reshape(N, cb, 128)
    p3 = pf.reshape(1, cb, 128)
    bt = 16
    while bt > 1 and N % bt:
        bt //= 2
    out = pl.pallas_call(
        functools.partial(_finalize_kernel, inv_hw=1.0 / HW),
        out_shape=jax.ShapeDtypeStruct((N, cb, 128), jnp.float32),
        grid=(N // bt,),
        in_specs=[
            pl.BlockSpec((bt, cb, 128), lambda b: (b, 0, 0)),
            pl.BlockSpec((1, cb, 128), lambda b: (0, 0, 0)),
        ],
        out_specs=pl.BlockSpec((bt, cb, 128), lambda b: (b, 0, 0)),
        compiler_params=pltpu.CompilerParams(
            dimension_semantics=("parallel",)),
    )(s3, p3)

    return out.reshape(N, C, 1, 1)


# speedup vs baseline: 3.3487x; 1.6519x over previous
import functools

import jax
import jax.numpy as jnp
from jax.experimental import pallas as pl
from jax.experimental.pallas import tpu as pltpu

_EPS = 1e-6
_LANES = 128


def _gem_kernel(x_ref, pcol_ref, prow_ref, o_ref, *, hw):
    xm = jnp.maximum(x_ref[...], _EPS)
    xp = jnp.exp2(jnp.log2(xm) * pcol_ref[...])
    s_col = jnp.sum(xp, axis=-1, keepdims=True)
    s_row = jax.lax.transpose(s_col, (1, 0))
    prow = prow_ref[...]
    pad_terms = (_LANES - hw) * jnp.exp2(jnp.log2(_EPS) * prow)
    m = (s_row - pad_terms) * (1.0 / hw)
    o_ref[...] = jnp.exp2(jnp.log2(m) * (1.0 / prow))[None]


def kernel(x, p):
    N, C, H, W = x.shape
    HW = H * W
    NC = N * C

    x2 = x.reshape(NC, HW).astype(jnp.float32)
    x_pad = jnp.pad(x2, ((0, 0), (0, _LANES - HW)), constant_values=0.0)
    pf = p.astype(jnp.float32)
    p_col = pf.reshape(C, 1)
    p_row = pf.reshape(1, C)

    out = pl.pallas_call(
        functools.partial(_gem_kernel, hw=HW),
        out_shape=jax.ShapeDtypeStruct((N, 1, C), jnp.float32),
        grid=(N,),
        in_specs=[
            pl.BlockSpec((C, _LANES), lambda n: (n, 0)),
            pl.BlockSpec((C, 1), lambda n: (0, 0)),
            pl.BlockSpec((1, C), lambda n: (0, 0)),
        ],
        out_specs=pl.BlockSpec((1, 1, C), lambda n: (n, 0, 0)),
        compiler_params=pltpu.CompilerParams(
            dimension_semantics=("parallel",)),
    )(x_pad, p_col, p_row)

    return out.reshape(N, C, 1, 1)
